# Initial kernel scaffold; baseline (speedup 1.0000x reference)
import functools

import jax
import jax.numpy as jnp
from jax import lax
from jax.experimental import pallas as pl
from jax.experimental.pallas import tpu as pltpu
from jax.experimental.pallas import tpu_sc as plsc

D_VOCAB = 100000
D_MODEL = 1024
BATCH = 4
POS = 2048

NC = 2
NS = 16
NW = NC * NS

B = BATCH * POS
B_PER_W = B // NW
CH = 32
N_CHUNKS = B_PER_W // CH


def _embed_gather(tokens_flat, w_e):
    mesh = plsc.VectorSubcoreMesh(core_axis_name="c", subcore_axis_name="s")
    idx = tokens_flat.reshape(NW, N_CHUNKS, CH)

    @functools.partial(
        pl.kernel,
        mesh=mesh,
        out_type=jax.ShapeDtypeStruct((B, D_MODEL), jnp.float32),
        scratch_types=[
            pltpu.VMEM((N_CHUNKS, CH), jnp.int32),
            pltpu.VMEM((CH, D_MODEL), jnp.float32),
            pltpu.VMEM((CH, D_MODEL), jnp.float32),
            pltpu.SemaphoreType.DMA,
            pltpu.SemaphoreType.DMA,
            pltpu.SemaphoreType.DMA,
            pltpu.SemaphoreType.DMA,
        ],
    )
    def k(table_hbm, idx_hbm, out_hbm, idx_v, rows_a, rows_b, gsem_a, gsem_b,
          wsem_a, wsem_b):
        wid = lax.axis_index("s") * NC + lax.axis_index("c")
        base = wid * B_PER_W
        pltpu.sync_copy(idx_hbm.at[wid], idx_v)

        bufs = ((rows_a, gsem_a, wsem_a), (rows_b, gsem_b, wsem_b))

        pltpu.async_copy(table_hbm.at[idx_v.at[0]], rows_a, gsem_a).start()

        @pl.loop(0, N_CHUNKS, step=2)
        def _(j):
            for p in range(2):
                rows, gsem, wsem = bufs[p]
                nrows, ngsem, _ = bufs[1 - p]
                cur = j + p
                nxt = cur + 1
                @pl.when(nxt < N_CHUNKS)
                def _():
                    pltpu.async_copy(
                        table_hbm.at[idx_v.at[nxt]], nrows, ngsem).start()
                pltpu.make_async_copy(
                    table_hbm.at[idx_v.at[cur]], rows, gsem).wait()
                pltpu.async_copy(
                    rows, out_hbm.at[pl.ds(base + cur * CH, CH)], wsem).start()
                pltpu.make_async_copy(
                    rows, out_hbm.at[pl.ds(base + cur * CH, CH)], wsem).wait()

        return None

    return k(w_e, idx)


def kernel(tokens, W_E):
    tokens_flat = tokens.reshape(B).astype(jnp.int32)
    out = _embed_gather(tokens_flat, W_E)
    return out.reshape(BATCH, POS, D_MODEL)


# speedup vs baseline: 1.5485x; 1.5485x over previous
import functools

import jax
import jax.numpy as jnp
from jax import lax
from jax.experimental import pallas as pl
from jax.experimental.pallas import tpu as pltpu
from jax.experimental.pallas import tpu_sc as plsc

D_VOCAB = 100000
D_MODEL = 1024
BATCH = 4
POS = 2048

NC = 2
NS = 16
NW = NC * NS

B = BATCH * POS
B_PER_W = B // NW
CH = 32
N_CHUNKS = B_PER_W // CH


def _embed_gather(tokens_flat, w_e):
    mesh = plsc.VectorSubcoreMesh(core_axis_name="c", subcore_axis_name="s")
    idx = tokens_flat.reshape(NW, N_CHUNKS, CH)

    @functools.partial(
        pl.kernel,
        mesh=mesh,
        out_type=jax.ShapeDtypeStruct((B, D_MODEL), jnp.float32),
        scratch_types=[
            pltpu.VMEM((N_CHUNKS, CH), jnp.int32),
            pltpu.VMEM((CH, D_MODEL), jnp.float32),
            pltpu.VMEM((CH, D_MODEL), jnp.float32),
            pltpu.SemaphoreType.DMA,
            pltpu.SemaphoreType.DMA,
            pltpu.SemaphoreType.DMA,
            pltpu.SemaphoreType.DMA,
        ],
    )
    def k(table_hbm, idx_hbm, out_hbm, idx_v, rows_a, rows_b, gsem_a, gsem_b,
          wsem_a, wsem_b):
        wid = lax.axis_index("s") * NC + lax.axis_index("c")
        base = wid * B_PER_W
        pltpu.sync_copy(idx_hbm.at[wid], idx_v)

        bufs = ((rows_a, gsem_a, wsem_a), (rows_b, gsem_b, wsem_b))

        pltpu.make_async_copy(table_hbm.at[idx_v.at[0]], rows_a, gsem_a).start()

        @pl.loop(0, N_CHUNKS, step=2)
        def _(j):
            for p in range(2):
                rows, gsem, wsem = bufs[p]
                nrows, ngsem, _ = bufs[1 - p]
                cur = j + p
                nxt = cur + 1
                @pl.when(nxt < N_CHUNKS)
                def _():
                    pltpu.make_async_copy(
                        table_hbm.at[idx_v.at[nxt]], nrows, ngsem).start()
                pltpu.make_async_copy(
                    table_hbm.at[idx_v.at[cur]], rows, gsem).wait()
                pltpu.make_async_copy(
                    rows, out_hbm.at[pl.ds(base + cur * CH, CH)], wsem).start()
                pltpu.make_async_copy(
                    rows, out_hbm.at[pl.ds(base + cur * CH, CH)], wsem).wait()

        return None

    return k(w_e, idx)


def kernel(tokens, W_E):
    tokens_flat = tokens.reshape(B).astype(jnp.int32)
    out = _embed_gather(tokens_flat, W_E)
    return out.reshape(BATCH, POS, D_MODEL)
